# baseline (speedup 1.0000x reference)
import numpy as np
import jax
import jax.numpy as jnp
from jax.experimental import pallas as pl

_VOCAB = 1000
_SCHEDULE = {}


def _fill_body(col_ref, out_ref):
    bt, v = out_ref.shape
    lane = jax.lax.broadcasted_iota(jnp.int32, (bt, v), 1)
    out_ref[...] = jnp.where(lane == col_ref[...], 10.0, -10.0)


def kernel(input_ids, anchor):
    B, T = input_ids.shape
    past_len = 0
    cols_np = np.array(
        [int(_SCHEDULE.get(past_len + t, 1)) for t in range(T)], dtype=np.int32
    )
    cols = jnp.asarray(np.tile(cols_np, B).reshape(B * T, 1))

    bt = 512
    rows = B * T
    out = pl.pallas_call(
        _fill_body,
        grid=(rows // bt,),
        in_specs=[pl.BlockSpec((bt, 1), lambda i: (i, 0))],
        out_specs=pl.BlockSpec((bt, _VOCAB), lambda i: (i, 0)),
        out_shape=jax.ShapeDtypeStruct((rows, _VOCAB), jnp.float32),
    )(cols)
    return out.reshape(B, T, _VOCAB)


# speedup vs baseline: 1.2387x; 1.2387x over previous
import functools

import numpy as np
import jax
import jax.numpy as jnp
from jax.experimental import pallas as pl

_VOCAB = 1000
_SCHEDULE = {}


def _uniform_body(col, out_ref):
    bt, v = out_ref.shape
    lane = jax.lax.broadcasted_iota(jnp.int32, (8, v), 1)
    rows8 = jnp.where(lane == col, 10.0, -10.0)
    out_ref[...] = jnp.broadcast_to(rows8[:1], (bt, v))


def _general_body(col_ref, out_ref):
    bt, v = out_ref.shape
    lane = jax.lax.broadcasted_iota(jnp.int32, (bt, v), 1)
    out_ref[...] = jnp.where(lane == col_ref[...], 10.0, -10.0)


def kernel(input_ids, anchor):
    B, T = input_ids.shape
    past_len = 0
    cols_np = np.array(
        [int(_SCHEDULE.get(past_len + t, 1)) for t in range(T)], dtype=np.int32
    )

    bt = 1024
    rows = B * T
    if bool((cols_np == cols_np[0]).all()):
        out = pl.pallas_call(
            functools.partial(_uniform_body, int(cols_np[0])),
            grid=(rows // bt,),
            out_specs=pl.BlockSpec((bt, _VOCAB), lambda i: (i, 0)),
            out_shape=jax.ShapeDtypeStruct((rows, _VOCAB), jnp.float32),
        )()
    else:
        cols = jnp.asarray(np.tile(cols_np, B).reshape(B * T, 1))
        out = pl.pallas_call(
            _general_body,
            grid=(rows // bt,),
            in_specs=[pl.BlockSpec((bt, 1), lambda i: (i, 0))],
            out_specs=pl.BlockSpec((bt, _VOCAB), lambda i: (i, 0)),
            out_shape=jax.ShapeDtypeStruct((rows, _VOCAB), jnp.float32),
        )(cols)
    return out.reshape(B, T, _VOCAB)
